# Initial kernel scaffold; baseline (speedup 1.0000x reference)
import jax
import jax.numpy as jnp
from jax.experimental import pallas as pl


def kernel(values, offsets, position_weight):
    raise NotImplementedError("write your pallas kernel here")


# speedup vs baseline: 5.0629x; 5.0629x over previous
import functools

import jax
import jax.numpy as jnp
from jax import lax
from jax.experimental import pallas as pl
from jax.experimental.pallas import tpu as pltpu
from jax.experimental.pallas import tpu_sc as plsc

_NUM_CORES = 2
_NUM_SUBCORES = 16
_LANES = 16
_NW = _NUM_CORES * _NUM_SUBCORES
_IDX_ROW = 128


@functools.partial(jax.jit, static_argnames=("n", "num_offsets"))
def _position_weights(offsets_padded, position_weight, n, num_offsets):
    chunk = n // _NW
    rows = chunk // _IDX_ROW
    vecs_per_row = _IDX_ROW // _LANES
    mesh = plsc.VectorSubcoreMesh(core_axis_name="c", subcore_axis_name="s")

    @functools.partial(
        pl.kernel,
        mesh=mesh,
        out_type=jax.ShapeDtypeStruct((n,), jnp.float32),
        scratch_types=[
            pltpu.VMEM((offsets_padded.shape[0],), jnp.int32),
            pltpu.VMEM((rows, _IDX_ROW), jnp.int32),
            pltpu.VMEM((rows, _IDX_ROW), jnp.float32),
            pltpu.SemaphoreType.DMA,
        ],
    )
    def body(offs_hbm, pw_hbm, out_hbm, offs_v, idx_v, w_v, sem):
        wid = lax.axis_index("s") * _NUM_CORES + lax.axis_index("c")
        base = wid * chunk
        pltpu.sync_copy(offs_hbm, offs_v)
        offs_vec = offs_v[pl.ds(0, _LANES)]
        interior = [offs_vec[k] for k in range(1, num_offsets - 1)]
        lane = lax.iota(jnp.int32, 16)

        for r in range(rows):
            for v in range(vecs_per_row):
                j = lane + (base + r * _IDX_ROW + v * _LANES)
                off = jnp.zeros((16,), jnp.int32)
                for ok in interior:
                    off = jnp.maximum(off, jnp.where(j >= ok, ok, 0))
                idx_v[r, v * _LANES:(v + 1) * _LANES] = j - off

        gathers = [
            pltpu.async_copy(pw_hbm.at[idx_v.at[r]], w_v.at[r], sem)
            for r in range(rows)
        ]
        for g in gathers:
            g.wait()
        for r in range(rows):
            pltpu.sync_copy(w_v.at[r], out_hbm.at[pl.ds(base + r * _IDX_ROW, _IDX_ROW)])

    return body(offsets_padded, position_weight)


def kernel(values, offsets, position_weight):
    n = values.shape[0]
    num_offsets = offsets.shape[0]
    pad = (-num_offsets) % 16
    offsets_padded = jnp.concatenate(
        [offsets, jnp.full((pad,), n, dtype=offsets.dtype)]
    )
    weights = _position_weights(offsets_padded, position_weight, n, num_offsets)
    return values, offsets, weights


# speedup vs baseline: 6.5897x; 1.3016x over previous
import functools

import jax
import jax.numpy as jnp
from jax import lax
from jax.experimental import pallas as pl
from jax.experimental.pallas import tpu as pltpu
from jax.experimental.pallas import tpu_sc as plsc

_NUM_CORES = 2
_NUM_SUBCORES = 16
_LANES = 16
_NW = _NUM_CORES * _NUM_SUBCORES


@functools.partial(jax.jit, static_argnames=("n",))
def _position_weights(offsets, position_weight, n):
    chunk = n // _NW
    vecs = chunk // _LANES
    mesh = plsc.VectorSubcoreMesh(core_axis_name="c", subcore_axis_name="s")

    @functools.partial(
        pl.kernel,
        mesh=mesh,
        out_type=jax.ShapeDtypeStruct((n,), jnp.float32),
        compiler_params=pltpu.CompilerParams(needs_layout_passes=False),
        scratch_types=[
            pltpu.VMEM((_LANES,), jnp.int32),
            pltpu.VMEM((n,), jnp.float32),
            pltpu.VMEM((chunk,), jnp.int32),
            pltpu.VMEM((chunk,), jnp.float32),
            pltpu.SemaphoreType.DMA,
        ],
    )
    def body(offs_hbm, pw_hbm, out_hbm, offs_v, pw_v, off_arr, out_v, sem):
        wid = lax.axis_index("s") * _NUM_CORES + lax.axis_index("c")
        base = wid * chunk
        table_dma = pltpu.async_copy(pw_hbm, pw_v, sem)
        pltpu.sync_copy(offs_hbm.at[pl.ds(0, _LANES)], offs_v)

        offs_vec = offs_v[...]
        zero = jnp.zeros((_LANES,), jnp.int32)
        for v in range(vecs):
            off_arr[v * _LANES:(v + 1) * _LANES] = zero
        carry = jnp.max(jnp.where(offs_vec <= base, offs_vec, 0))
        in_tile = (offs_vec > base) & (offs_vec < base + chunk)
        plsc.store_scatter(off_arr, [offs_vec - base], offs_vec, mask=in_tile)

        lane = lax.iota(jnp.int32, _LANES)
        table_dma.wait()
        for v in range(vecs):
            sl = slice(v * _LANES, (v + 1) * _LANES)
            off = jnp.maximum(plsc.cummax(off_arr[sl]), carry)
            carry = off[_LANES - 1]
            seq = lane + (base + v * _LANES) - off
            out_v[sl] = plsc.load_gather(pw_v, [seq])

        pltpu.sync_copy(out_v, out_hbm.at[pl.ds(base, chunk)])

    return body(offsets, position_weight)


def kernel(values, offsets, position_weight):
    n = values.shape[0]
    weights = _position_weights(offsets, position_weight, n)
    return values, offsets, weights


# speedup vs baseline: 6.6169x; 1.0041x over previous
import functools

import jax
import jax.numpy as jnp
from jax import lax
from jax.experimental import pallas as pl
from jax.experimental.pallas import tpu as pltpu
from jax.experimental.pallas import tpu_sc as plsc

_NUM_CORES = 2
_NUM_SUBCORES = 16
_LANES = 16
_NW = _NUM_CORES * _NUM_SUBCORES


@functools.partial(jax.jit, static_argnames=("n",))
def _position_weights(offsets, position_weight, n):
    chunk = n // _NW
    vecs = chunk // _LANES
    mesh = plsc.VectorSubcoreMesh(core_axis_name="c", subcore_axis_name="s")

    @functools.partial(
        pl.kernel,
        mesh=mesh,
        out_type=jax.ShapeDtypeStruct((n,), jnp.float32),
        compiler_params=pltpu.CompilerParams(needs_layout_passes=False),
        scratch_types=[
            pltpu.VMEM((_LANES,), jnp.int32),
            pltpu.VMEM((n,), jnp.float32),
            pltpu.VMEM((chunk,), jnp.int32),
            pltpu.VMEM((chunk,), jnp.float32),
            pltpu.SemaphoreType.DMA,
        ],
    )
    def body(offs_hbm, pw_hbm, out_hbm, offs_v, pw_v, off_arr, out_v, sem):
        wid = lax.axis_index("s") * _NUM_CORES + lax.axis_index("c")
        base = wid * chunk
        table_dma = pltpu.async_copy(pw_hbm, pw_v, sem)
        pltpu.sync_copy(offs_hbm.at[pl.ds(0, _LANES)], offs_v)

        offs_vec = offs_v[...]
        zero = jnp.zeros((_LANES,), jnp.int32)

        def zero_step(v, carry):
            off_arr[pl.ds(v * _LANES, _LANES)] = zero
            return carry

        lax.fori_loop(0, vecs, zero_step, 0)
        carry0 = jnp.max(jnp.where(offs_vec <= base, offs_vec, 0))
        in_tile = (offs_vec > base) & (offs_vec < base + chunk)
        plsc.store_scatter(off_arr, [offs_vec - base], offs_vec, mask=in_tile)

        lane = lax.iota(jnp.int32, _LANES)
        table_dma.wait()

        def step(v, carry):
            start = v * _LANES
            off = jnp.maximum(plsc.cummax(off_arr[pl.ds(start, _LANES)]), carry)
            seq = lane + (base + start) - off
            out_v[pl.ds(start, _LANES)] = plsc.load_gather(pw_v, [seq])
            return off[_LANES - 1]

        lax.fori_loop(0, vecs, step, carry0)
        pltpu.sync_copy(out_v, out_hbm.at[pl.ds(base, chunk)])

    return body(offsets, position_weight)


def kernel(values, offsets, position_weight):
    n = values.shape[0]
    weights = _position_weights(offsets, position_weight, n)
    return values, offsets, weights
